# baseline (speedup 1.0000x reference)
import functools

import jax
import jax.numpy as jnp
from jax import lax
from jax.experimental import pallas as pl
from jax.experimental.pallas import tpu as pltpu
from jax.sharding import Mesh, PartitionSpec as P

E = 10
TOPK = 3
B = 1024
D_IN = 1024
D_EXP = 512
T = 2
NEG = -1e30


def _mmoe_kernel(bloc, x_ref, We1_ref, be1_ref, We2_ref, be2_ref,
                 Wg_ref, Wn_ref, noise_ref, Wt1_ref, bt1_ref, Wt2_ref,
                 bt2_ref, out0_ref, out1_ref, acc_ref, g_ref, xb_ref):
    e = pl.program_id(0)

    @pl.when(e == 0)
    def _prologue():
        x = x_ref[...]
        xb_ref[...] = x.astype(jnp.bfloat16)
        gsum = jnp.zeros((bloc, E), jnp.float32)
        iota = lax.broadcasted_iota(jnp.int32, (bloc, E), 1)
        for i in range(T):
            mean = jnp.dot(x, Wg_ref[i], preferred_element_type=jnp.float32)
            std = jax.nn.softplus(
                jnp.dot(x, Wn_ref[i], preferred_element_type=jnp.float32))
            H = mean + noise_ref[i] * std
            Hw = H
            for _ in range(TOPK - 1):
                m = jnp.max(Hw, axis=1, keepdims=True)
                idx = jnp.min(jnp.where(Hw == m, iota, E), axis=1,
                              keepdims=True)
                Hw = jnp.where(iota == idx, NEG, Hw)
            thresh = jnp.max(Hw, axis=1, keepdims=True)
            Hm = jnp.where(H < thresh, NEG, H)
            mx = jnp.max(Hm, axis=1, keepdims=True)
            p = jnp.exp(Hm - mx)
            gsum = gsum + p / jnp.sum(p, axis=1, keepdims=True)
        g_ref[...] = gsum

    onehot = (lax.broadcasted_iota(jnp.int32, (E, 1), 0) == e).astype(
        jnp.float32)
    gcol = jnp.dot(g_ref[...], onehot, preferred_element_type=jnp.float32)
    We1 = We1_ref[0]
    We2 = We2_ref[0]
    be1 = be1_ref[0]
    be2 = be2_ref[0]
    half = bloc // 2
    for s in range(2):
        rows = slice(s * half, (s + 1) * half)
        h = jnp.maximum(
            jnp.dot(xb_ref[rows, :], We1.astype(jnp.bfloat16),
                    preferred_element_type=jnp.float32) + be1, 0.0)
        f = (jnp.dot(h.astype(jnp.bfloat16), We2.astype(jnp.bfloat16),
                     preferred_element_type=jnp.float32) + be2)
        contrib = gcol[rows, :] * f

        @pl.when(e == 0)
        def _init():
            acc_ref[rows, :] = contrib

        @pl.when(e > 0)
        def _accum():
            acc_ref[rows, :] += contrib

    @pl.when(e == E - 1)
    def _heads():
        shared = acc_ref[...].astype(jnp.bfloat16)
        for t, out_ref in ((0, out0_ref), (1, out1_ref)):
            ht = jnp.maximum(
                jnp.dot(shared, Wt1_ref[t].astype(jnp.bfloat16),
                        preferred_element_type=jnp.float32)
                + bt1_ref[t], 0.0)
            out_ref[...] = (
                jnp.dot(ht.astype(jnp.bfloat16),
                        Wt2_ref[t].astype(jnp.bfloat16),
                        preferred_element_type=jnp.float32)
                + bt2_ref[t])


def _one_core(bloc, x, We1, be1, We2, be2, Wg, Wn, noise, Wt1, bt1, Wt2,
              bt2):
    full = lambda s: pl.BlockSpec(s, lambda e: tuple(0 for _ in s))
    in_specs = [
        full((bloc, D_IN)),
        pl.BlockSpec((1, D_IN, D_EXP), lambda e: (e, 0, 0)),
        pl.BlockSpec((1, 1, D_EXP), lambda e: (e, 0, 0)),
        pl.BlockSpec((1, D_EXP, D_EXP), lambda e: (e, 0, 0)),
        pl.BlockSpec((1, 1, D_EXP), lambda e: (e, 0, 0)),
        full((T, D_IN, E)),
        full((T, D_IN, E)),
        full((T, bloc, E)),
        full((T, D_EXP, 512)),
        full((T, 512)),
        full((T, 512, 256)),
        full((T, 256)),
    ]
    out_specs = (full((bloc, 256)), full((bloc, 256)))
    return pl.pallas_call(
        functools.partial(_mmoe_kernel, bloc),
        grid=(E,),
        in_specs=in_specs,
        out_specs=out_specs,
        out_shape=(jax.ShapeDtypeStruct((bloc, 256), jnp.float32),
                   jax.ShapeDtypeStruct((bloc, 256), jnp.float32)),
        scratch_shapes=[pltpu.VMEM((bloc, D_EXP), jnp.float32),
                        pltpu.VMEM((bloc, E), jnp.float32),
                        pltpu.VMEM((bloc, D_IN), jnp.bfloat16)],
        compiler_params=pltpu.CompilerParams(
            dimension_semantics=("arbitrary",)),
    )(x, We1, be1, We2, be2, Wg, Wn, noise, Wt1, bt1, Wt2, bt2)


@jax.jit
def kernel(x, We1, be1, We2, be2, Wg, Wn, Wt1, bt1, Wt2, bt2):
    with jax.ensure_compile_time_eval():
        nkey = jax.random.key(42)
        noise = jnp.stack([
            jax.random.normal(jax.random.fold_in(nkey, i), (B, E),
                              dtype=jnp.float32)
            for i in range(T)])

    be1 = be1.reshape(E, 1, D_EXP)
    be2 = be2.reshape(E, 1, D_EXP)

    devs = [d for d in jax.devices() if d.platform == "tpu"]
    ndev = 1
    for n in (8, 4, 2):
        if len(devs) >= n and B % n == 0:
            ndev = n
            break
    if ndev == 1:
        return _one_core(B, x, We1, be1, We2, be2, Wg, Wn, noise, Wt1, bt1,
                         Wt2, bt2)

    mesh = Mesh(devs[:ndev], ("d",))
    rep = P()
    fn = jax.shard_map(
        functools.partial(_one_core, B // ndev),
        mesh=mesh,
        in_specs=(P("d", None), rep, rep, rep, rep, rep, rep,
                  P(None, "d", None), rep, rep, rep, rep),
        out_specs=(P("d", None), P("d", None)),
        check_vma=False,
    )
    return fn(x, We1, be1, We2, be2, Wg, Wn, noise, Wt1, bt1, Wt2, bt2)


# speedup vs baseline: 8.3156x; 8.3156x over previous
import functools

import jax
import jax.numpy as jnp
from jax import lax
from jax.experimental import pallas as pl
from jax.experimental.pallas import tpu as pltpu

E = 10
TOPK = 3
B = 1024
D_IN = 1024
D_EXP = 512
T = 2
NEG = -1e30


def _mmoe_kernel(bloc, x_ref, We1_ref, be1_ref, We2_ref, be2_ref,
                 Wg_ref, Wn_ref, noise_ref, Wt1_ref, bt1_ref, Wt2_ref,
                 bt2_ref, out0_ref, out1_ref, acc_ref, g_ref, xb_ref):
    e = pl.program_id(0)

    @pl.when(e == 0)
    def _prologue():
        x = x_ref[...]
        xb_ref[...] = x.astype(jnp.bfloat16)
        gsum = jnp.zeros((bloc, E), jnp.float32)
        iota = lax.broadcasted_iota(jnp.int32, (bloc, E), 1)
        for i in range(T):
            mean = jnp.dot(x, Wg_ref[i], preferred_element_type=jnp.float32)
            std = jax.nn.softplus(
                jnp.dot(x, Wn_ref[i], preferred_element_type=jnp.float32))
            H = mean + noise_ref[i] * std
            Hw = H
            for _ in range(TOPK - 1):
                m = jnp.max(Hw, axis=1, keepdims=True)
                idx = jnp.min(jnp.where(Hw == m, iota, E), axis=1,
                              keepdims=True)
                Hw = jnp.where(iota == idx, NEG, Hw)
            thresh = jnp.max(Hw, axis=1, keepdims=True)
            Hm = jnp.where(H < thresh, NEG, H)
            mx = jnp.max(Hm, axis=1, keepdims=True)
            p = jnp.exp(Hm - mx)
            gsum = gsum + p / jnp.sum(p, axis=1, keepdims=True)
        g_ref[...] = gsum

    onehot = (lax.broadcasted_iota(jnp.int32, (E, 1), 0) == e).astype(
        jnp.float32)
    gcol = jnp.dot(g_ref[...], onehot, preferred_element_type=jnp.float32)
    We1 = We1_ref[0]
    We2 = We2_ref[0]
    be1 = be1_ref[0]
    be2 = be2_ref[0]
    nsplit = 4
    half = bloc // nsplit
    for s in range(nsplit):
        rows = slice(s * half, (s + 1) * half)
        h = jnp.maximum(
            jnp.dot(xb_ref[rows, :], We1.astype(jnp.bfloat16),
                    preferred_element_type=jnp.float32) + be1, 0.0)
        f = (jnp.dot(h.astype(jnp.bfloat16), We2.astype(jnp.bfloat16),
                     preferred_element_type=jnp.float32) + be2)
        contrib = gcol[rows, :] * f

        @pl.when(e == 0)
        def _init():
            acc_ref[rows, :] = contrib

        @pl.when(e > 0)
        def _accum():
            acc_ref[rows, :] += contrib

    @pl.when(e == E - 1)
    def _heads():
        shared = acc_ref[...].astype(jnp.bfloat16)
        for t, out_ref in ((0, out0_ref), (1, out1_ref)):
            ht = jnp.maximum(
                jnp.dot(shared, Wt1_ref[t].astype(jnp.bfloat16),
                        preferred_element_type=jnp.float32)
                + bt1_ref[t], 0.0)
            out_ref[...] = (
                jnp.dot(ht.astype(jnp.bfloat16),
                        Wt2_ref[t].astype(jnp.bfloat16),
                        preferred_element_type=jnp.float32)
                + bt2_ref[t])


def _one_core(bloc, x, We1, be1, We2, be2, Wg, Wn, noise, Wt1, bt1, Wt2,
              bt2):
    full = lambda s: pl.BlockSpec(s, lambda e: tuple(0 for _ in s))
    in_specs = [
        full((bloc, D_IN)),
        pl.BlockSpec((1, D_IN, D_EXP), lambda e: (e, 0, 0)),
        pl.BlockSpec((1, 1, D_EXP), lambda e: (e, 0, 0)),
        pl.BlockSpec((1, D_EXP, D_EXP), lambda e: (e, 0, 0)),
        pl.BlockSpec((1, 1, D_EXP), lambda e: (e, 0, 0)),
        full((T, D_IN, E)),
        full((T, D_IN, E)),
        full((T, bloc, E)),
        full((T, D_EXP, 512)),
        full((T, 512)),
        full((T, 512, 256)),
        full((T, 256)),
    ]
    out_specs = (full((bloc, 256)), full((bloc, 256)))
    return pl.pallas_call(
        functools.partial(_mmoe_kernel, bloc),
        grid=(E,),
        in_specs=in_specs,
        out_specs=out_specs,
        out_shape=(jax.ShapeDtypeStruct((bloc, 256), jnp.float32),
                   jax.ShapeDtypeStruct((bloc, 256), jnp.float32)),
        scratch_shapes=[pltpu.VMEM((bloc, D_EXP), jnp.float32),
                        pltpu.VMEM((bloc, E), jnp.float32),
                        pltpu.VMEM((bloc, D_IN), jnp.bfloat16)],
        compiler_params=pltpu.CompilerParams(
            dimension_semantics=("arbitrary",)),
    )(x, We1, be1, We2, be2, Wg, Wn, noise, Wt1, bt1, Wt2, bt2)


@jax.jit
def kernel(x, We1, be1, We2, be2, Wg, Wn, Wt1, bt1, Wt2, bt2):
    with jax.ensure_compile_time_eval():
        nkey = jax.random.key(42)
        noise = jnp.stack([
            jax.random.normal(jax.random.fold_in(nkey, i), (B, E),
                              dtype=jnp.float32)
            for i in range(T)])

    be1 = be1.reshape(E, 1, D_EXP)
    be2 = be2.reshape(E, 1, D_EXP)
    return _one_core(B, x, We1, be1, We2, be2, Wg, Wn, noise, Wt1, bt1,
                     Wt2, bt2)


# speedup vs baseline: 12.4751x; 1.5002x over previous
import jax
import jax.numpy as jnp
from jax import lax
from jax.experimental import pallas as pl
from jax.experimental.pallas import tpu as pltpu

E = 10
TOPK = 3
B = 1024
D_IN = 1024
D_EXP = 512
T = 2
NEG = -1e30


def _mmoe_kernel(x_ref, We1_ref, be1_ref, We2_ref, be2_ref,
                 Wg_ref, Wn_ref, noise_ref, Wt1_ref, bt1_ref, Wt2_ref,
                 bt2_ref, out0_ref, out1_ref, w1buf, w2buf, sem1, sem2):
    def start_copy(e, slot):
        pltpu.make_async_copy(We1_ref.at[e], w1buf.at[slot],
                              sem1.at[slot]).start()
        pltpu.make_async_copy(We2_ref.at[e], w2buf.at[slot],
                              sem2.at[slot]).start()

    def wait_copy(e, slot):
        pltpu.make_async_copy(We1_ref.at[e], w1buf.at[slot],
                              sem1.at[slot]).wait()
        pltpu.make_async_copy(We2_ref.at[e], w2buf.at[slot],
                              sem2.at[slot]).wait()

    start_copy(0, 0)
    start_copy(1, 1)

    x = x_ref[...]
    xb = x.astype(jnp.bfloat16)

    gsum = jnp.zeros((B, E), jnp.float32)
    iota = lax.broadcasted_iota(jnp.int32, (B, E), 1)
    for i in range(T):
        mean = jnp.dot(x, Wg_ref[i], preferred_element_type=jnp.float32)
        std = jax.nn.softplus(
            jnp.dot(x, Wn_ref[i], preferred_element_type=jnp.float32))
        H = mean + noise_ref[i] * std
        Hw = H
        for _ in range(TOPK - 1):
            m = jnp.max(Hw, axis=1, keepdims=True)
            idx = jnp.min(jnp.where(Hw == m, iota, E), axis=1, keepdims=True)
            Hw = jnp.where(iota == idx, NEG, Hw)
        thresh = jnp.max(Hw, axis=1, keepdims=True)
        Hm = jnp.where(H < thresh, NEG, H)
        mx = jnp.max(Hm, axis=1, keepdims=True)
        p = jnp.exp(Hm - mx)
        gsum = gsum + p / jnp.sum(p, axis=1, keepdims=True)

    acc = jnp.zeros((B, D_EXP), jnp.float32)
    for e in range(E):
        slot = e % 2
        wait_copy(e, slot)
        We1 = w1buf[slot].astype(jnp.bfloat16)
        We2 = w2buf[slot].astype(jnp.bfloat16)
        h = jnp.maximum(
            jnp.dot(xb, We1, preferred_element_type=jnp.float32)
            + be1_ref[e], 0.0)
        f = (jnp.dot(h.astype(jnp.bfloat16), We2,
                     preferred_element_type=jnp.float32) + be2_ref[e])
        acc = acc + gsum[:, e:e + 1] * f
        if e + 2 < E:
            start_copy(e + 2, slot)

    shared = acc.astype(jnp.bfloat16)
    for t, out_ref in ((0, out0_ref), (1, out1_ref)):
        ht = jnp.maximum(
            jnp.dot(shared, Wt1_ref[t].astype(jnp.bfloat16),
                    preferred_element_type=jnp.float32) + bt1_ref[t], 0.0)
        out_ref[...] = (
            jnp.dot(ht.astype(jnp.bfloat16), Wt2_ref[t].astype(jnp.bfloat16),
                    preferred_element_type=jnp.float32) + bt2_ref[t])


@jax.jit
def kernel(x, We1, be1, We2, be2, Wg, Wn, Wt1, bt1, Wt2, bt2):
    with jax.ensure_compile_time_eval():
        nkey = jax.random.key(42)
        noise = jnp.stack([
            jax.random.normal(jax.random.fold_in(nkey, i), (B, E),
                              dtype=jnp.float32)
            for i in range(T)])

    vmem = pl.BlockSpec(memory_space=pltpu.MemorySpace.VMEM)
    hbm = pl.BlockSpec(memory_space=pl.MemorySpace.ANY)
    out0, out1 = pl.pallas_call(
        _mmoe_kernel,
        in_specs=[vmem, hbm, vmem, hbm, vmem, vmem, vmem, vmem, vmem, vmem,
                  vmem, vmem],
        out_specs=(vmem, vmem),
        out_shape=(jax.ShapeDtypeStruct((B, 256), jnp.float32),
                   jax.ShapeDtypeStruct((B, 256), jnp.float32)),
        scratch_shapes=[pltpu.VMEM((2, D_IN, D_EXP), jnp.float32),
                        pltpu.VMEM((2, D_EXP, D_EXP), jnp.float32),
                        pltpu.SemaphoreType.DMA((2,)),
                        pltpu.SemaphoreType.DMA((2,))],
    )(x, We1, be1, We2, be2, Wg, Wn, noise, Wt1, bt1, Wt2, bt2)
    return (out0, out1)


# speedup vs baseline: 12.6077x; 1.0106x over previous
import jax
import jax.numpy as jnp
from jax import lax
from jax.experimental import pallas as pl
from jax.experimental.pallas import tpu as pltpu

E = 10
TOPK = 3
B = 1024
D_IN = 1024
D_EXP = 512
T = 2
NEG = -1e30


def _mmoe_kernel(x_ref, We1_ref, be1_ref, We2_ref, be2_ref,
                 Wg_ref, Wn_ref, noise_ref, Wt1_ref, bt1_ref, Wt2_ref,
                 bt2_ref, out0_ref, out1_ref, w1buf, w2buf, sem1, sem2):
    def start_copy(e, slot):
        pltpu.make_async_copy(We1_ref.at[e], w1buf.at[slot],
                              sem1.at[slot]).start()
        pltpu.make_async_copy(We2_ref.at[e], w2buf.at[slot],
                              sem2.at[slot]).start()

    def wait_copy(e, slot):
        pltpu.make_async_copy(We1_ref.at[e], w1buf.at[slot],
                              sem1.at[slot]).wait()
        pltpu.make_async_copy(We2_ref.at[e], w2buf.at[slot],
                              sem2.at[slot]).wait()

    start_copy(0, 0)
    start_copy(1, 1)

    x = x_ref[...]

    gsum = jnp.zeros((B, E), jnp.float32)
    iota = lax.broadcasted_iota(jnp.int32, (B, E), 1)
    for i in range(T):
        mean = jnp.dot(x, Wg_ref[i], preferred_element_type=jnp.float32)
        std = jax.nn.softplus(
            jnp.dot(x, Wn_ref[i], preferred_element_type=jnp.float32))
        H = mean + noise_ref[i] * std
        Hw = H
        for _ in range(TOPK - 1):
            m = jnp.max(Hw, axis=1, keepdims=True)
            idx = jnp.min(jnp.where(Hw == m, iota, E), axis=1, keepdims=True)
            Hw = jnp.where(iota == idx, NEG, Hw)
        thresh = jnp.max(Hw, axis=1, keepdims=True)
        Hm = jnp.where(H < thresh, NEG, H)
        mx = jnp.max(Hm, axis=1, keepdims=True)
        p = jnp.exp(Hm - mx)
        gsum = gsum + p / jnp.sum(p, axis=1, keepdims=True)

    acc = jnp.zeros((B, D_EXP), jnp.float32)
    for e in range(E):
        slot = e % 2
        wait_copy(e, slot)
        h = jnp.maximum(
            jnp.dot(x, w1buf[slot], preferred_element_type=jnp.float32)
            + be1_ref[e], 0.0)
        f = (jnp.dot(h, w2buf[slot],
                     preferred_element_type=jnp.float32) + be2_ref[e])
        acc = acc + gsum[:, e:e + 1] * f
        if e + 2 < E:
            start_copy(e + 2, slot)

    for t, out_ref in ((0, out0_ref), (1, out1_ref)):
        ht = jnp.maximum(
            jnp.dot(acc, Wt1_ref[t],
                    preferred_element_type=jnp.float32) + bt1_ref[t], 0.0)
        out_ref[...] = (
            jnp.dot(ht, Wt2_ref[t],
                    preferred_element_type=jnp.float32) + bt2_ref[t])


@jax.jit
def kernel(x, We1, be1, We2, be2, Wg, Wn, Wt1, bt1, Wt2, bt2):
    with jax.ensure_compile_time_eval():
        nkey = jax.random.key(42)
        noise = jnp.stack([
            jax.random.normal(jax.random.fold_in(nkey, i), (B, E),
                              dtype=jnp.float32)
            for i in range(T)])

    vmem = pl.BlockSpec(memory_space=pltpu.MemorySpace.VMEM)
    hbm = pl.BlockSpec(memory_space=pl.MemorySpace.ANY)
    out0, out1 = pl.pallas_call(
        _mmoe_kernel,
        in_specs=[vmem, hbm, vmem, hbm, vmem, vmem, vmem, vmem, vmem, vmem,
                  vmem, vmem],
        out_specs=(vmem, vmem),
        out_shape=(jax.ShapeDtypeStruct((B, 256), jnp.float32),
                   jax.ShapeDtypeStruct((B, 256), jnp.float32)),
        scratch_shapes=[pltpu.VMEM((2, D_IN, D_EXP), jnp.float32),
                        pltpu.VMEM((2, D_EXP, D_EXP), jnp.float32),
                        pltpu.SemaphoreType.DMA((2,)),
                        pltpu.SemaphoreType.DMA((2,))],
    )(x, We1, be1, We2, be2, Wg, Wn, noise, Wt1, bt1, Wt2, bt2)
    return (out0, out1)
